# baseline (speedup 1.0000x reference)
import functools

import jax
import jax.numpy as jnp
from jax import lax
from jax.experimental import pallas as pl
from jax.experimental.pallas import tpu as pltpu
from jax.experimental.pallas import tpu_sc as plsc

NBUF = 4
LANES = 16


def _build(NW, L, B, E, V):
  mesh = plsc.VectorSubcoreMesh(core_axis_name="c", subcore_axis_name="s")
  NC = plsc.get_sparse_core_info().num_cores
  G = B // NW
  EV = E // LANES

  @functools.partial(
      pl.kernel,
      out_type=jax.ShapeDtypeStruct((L, E, B), jnp.float32),
      mesh=mesh,
      compiler_params=pltpu.CompilerParams(use_tc_tiling_on_sc=True,
                                           needs_layout_passes=False),
      scratch_types=[
          pltpu.VMEM((L, G), jnp.int32),
          pltpu.VMEM((NBUF, G), jnp.int32),
          pltpu.VMEM((NBUF, G, 128), jnp.float32),
          pltpu.VMEM((NBUF, E, G), jnp.float32),
          pltpu.VMEM((NBUF, 128), jnp.float32),
          pltpu.SemaphoreType.DMA((NBUF,)),
          pltpu.SemaphoreType.DMA((NBUF,)),
          pltpu.SemaphoreType.DMA((NBUF,)),
      ],
  )
  def k(idx_hbm, tok2_hbm, poss_hbm, out_hbm, idx_v, idx2, rows, chunk,
        posb, sem_p, sem_b, sem_c):
    c = lax.axis_index("c")
    s = lax.axis_index("s")
    wid = s * NC + c
    b0 = wid * G

    pltpu.sync_copy(idx_hbm.at[:, pl.ds(b0, G)], idx_v)

    def step(j, carry):
      @pl.when(j < L)
      def _():
        r = lax.rem(j, NBUF)
        @pl.when(j >= NBUF)
        def _():
          pltpu.make_async_copy(chunk.at[r], out_hbm.at[0, :, pl.ds(0, G)],
                                sem_c.at[r]).wait()
        pltpu.async_copy(poss_hbm.at[j], posb.at[r], sem_p.at[r])
        for k8 in range(G // LANES):
          d = pl.ds(k8 * LANES, LANES)
          idx2[r, d] = lax.shift_right_logical(idx_v[j, d], 1)
        pltpu.async_copy(tok2_hbm.at[idx2.at[r]], rows.at[r], sem_b.at[r])

      jb = j - 1
      @pl.when((jb >= 0) & (jb < L))
      def _():
        r = lax.rem(jb, NBUF)
        pltpu.make_async_copy(poss_hbm.at[0], posb.at[r], sem_p.at[r]).wait()
        pltpu.make_async_copy(tok2_hbm.at[idx2.at[r]], rows.at[r],
                              sem_b.at[r]).wait()
        iot = lax.iota(jnp.int32, LANES)
        zero = jnp.zeros((LANES,), jnp.int32)
        base = [
            (iot + (k8 * LANES)) * 128
            + (idx_v[jb, pl.ds(k8 * LANES, LANES)] & 1) * E
            for k8 in range(G // LANES)
        ]
        bvec = [iot + (k8 * LANES) for k8 in range(G // LANES)]

        @plsc.parallel_loop(0, LANES, 1, unroll=8)
        def _(d):
          rotd = (iot + d) & (LANES - 1)
          for e16 in range(E // LANES):
            ecol = rotd + (e16 * LANES)
            pvec = plsc.load_gather(posb.at[r], [ecol])
            for k8 in range(G // LANES):
              val = plsc.load_gather(rows.at[r], [zero, base[k8] + ecol])
              plsc.store_scatter(chunk.at[r], [ecol, bvec[k8]], val + pvec)

        pltpu.async_copy(chunk.at[r], out_hbm.at[jb, :, pl.ds(b0, G)],
                         sem_c.at[r])

      return carry

    lax.fori_loop(0, L + 1, step, 0)

    for rb in range(NBUF):
      pltpu.make_async_copy(chunk.at[rb], out_hbm.at[0, :, pl.ds(0, G)],
                            sem_c.at[rb]).wait()

  return k


def kernel(in_idx, tok_table, pos_table):
  B, L = in_idx.shape
  V, E = tok_table.shape
  info = plsc.get_sparse_core_info()
  NW = info.num_cores * info.num_subcores
  assert B % (NW * 128) == 0 or B == NW * 128

  idxT = in_idx.T.astype(jnp.int32)
  tok2 = tok_table.reshape(V // 2, 2 * E)
  poss = jnp.pad(pos_table[:L], ((0, 0), (0, 128 - E)))
  out5 = _build(NW, L, B, E, V)(idxT, tok2, poss)
  return out5.transpose(2, 0, 1)


# speedup vs baseline: 1.0045x; 1.0045x over previous
import functools

import jax
import jax.numpy as jnp
from jax import lax
from jax.experimental import pallas as pl
from jax.experimental.pallas import tpu as pltpu
from jax.experimental.pallas import tpu_sc as plsc

NBUF = 4
LANES = 16


def _build(NW, L, B, E, V):
  mesh = plsc.VectorSubcoreMesh(core_axis_name="c", subcore_axis_name="s")
  NC = plsc.get_sparse_core_info().num_cores
  G = B // NW
  EV = E // LANES

  @functools.partial(
      pl.kernel,
      out_type=jax.ShapeDtypeStruct((L, E, B), jnp.float32),
      mesh=mesh,
      compiler_params=pltpu.CompilerParams(use_tc_tiling_on_sc=True,
                                           needs_layout_passes=False),
      scratch_types=[
          pltpu.VMEM((L, G), jnp.int32),
          pltpu.VMEM((NBUF, G), jnp.int32),
          pltpu.VMEM((NBUF, G, 128), jnp.float32),
          pltpu.VMEM((NBUF, E, G), jnp.float32),
          pltpu.VMEM((NBUF, 128), jnp.float32),
          pltpu.SemaphoreType.DMA((NBUF,)),
          pltpu.SemaphoreType.DMA((NBUF,)),
          pltpu.SemaphoreType.DMA((NBUF,)),
      ],
  )
  def k(idx_hbm, tok2_hbm, poss_hbm, out_hbm, idx_v, idx2, rows, chunk,
        posb, sem_p, sem_b, sem_c):
    c = lax.axis_index("c")
    s = lax.axis_index("s")
    wid = s * NC + c
    b0 = wid * G

    pltpu.sync_copy(idx_hbm.at[:, pl.ds(b0, G)], idx_v)

    def step(j, carry):
      @pl.when(j < L)
      def _():
        r = lax.rem(j, NBUF)
        @pl.when(j >= NBUF)
        def _():
          pltpu.make_async_copy(chunk.at[r], out_hbm.at[0, :, pl.ds(0, G)],
                                sem_c.at[r]).wait()
        pltpu.async_copy(poss_hbm.at[j], posb.at[r], sem_p.at[r])
        for k8 in range(G // LANES):
          d = pl.ds(k8 * LANES, LANES)
          idx2[r, d] = lax.shift_right_logical(idx_v[j, d], 1)
        pltpu.async_copy(tok2_hbm.at[idx2.at[r]], rows.at[r], sem_b.at[r])

      jb = j - 1
      @pl.when((jb >= 0) & (jb < L))
      def _():
        r = lax.rem(jb, NBUF)
        pltpu.make_async_copy(poss_hbm.at[0], posb.at[r], sem_p.at[r]).wait()
        pltpu.make_async_copy(tok2_hbm.at[idx2.at[r]], rows.at[r],
                              sem_b.at[r]).wait()
        iot = lax.iota(jnp.int32, LANES)
        zero = jnp.zeros((LANES,), jnp.int32)
        base = [
            (iot + (k8 * LANES)) * 128
            + (idx_v[jb, pl.ds(k8 * LANES, LANES)] & 1) * E
            for k8 in range(G // LANES)
        ]
        bvec = [iot + (k8 * LANES) for k8 in range(G // LANES)]

        @plsc.parallel_loop(0, LANES, 1, unroll=4)
        def _(d):
          rotd = (iot + d) & (LANES - 1)
          for e16 in range(E // LANES):
            ecol = rotd + (e16 * LANES)
            pvec = plsc.load_gather(posb.at[r], [ecol])
            for k8 in range(G // LANES):
              val = plsc.load_gather(rows.at[r], [zero, base[k8] + ecol])
              plsc.store_scatter(chunk.at[r], [ecol, bvec[k8]], val + pvec)

        pltpu.async_copy(chunk.at[r], out_hbm.at[jb, :, pl.ds(b0, G)],
                         sem_c.at[r])

      return carry

    lax.fori_loop(0, L + 1, step, 0)

    for rb in range(NBUF):
      pltpu.make_async_copy(chunk.at[rb], out_hbm.at[0, :, pl.ds(0, G)],
                            sem_c.at[rb]).wait()

  return k


def kernel(in_idx, tok_table, pos_table):
  B, L = in_idx.shape
  V, E = tok_table.shape
  info = plsc.get_sparse_core_info()
  NW = info.num_cores * info.num_subcores
  assert B % (NW * 128) == 0 or B == NW * 128

  idxT = in_idx.T.astype(jnp.int32)
  tok2 = tok_table.reshape(V // 2, 2 * E)
  poss = jnp.pad(pos_table[:L], ((0, 0), (0, 128 - E)))
  out5 = _build(NW, L, B, E, V)(idxT, tok2, poss)
  return out5.transpose(2, 0, 1)


# speedup vs baseline: 1.0424x; 1.0377x over previous
import functools

import jax
import jax.numpy as jnp
from jax import lax
from jax.experimental import pallas as pl
from jax.experimental.pallas import tpu as pltpu
from jax.experimental.pallas import tpu_sc as plsc

NBUF = 4
LANES = 16


def _build(NW, L, B, E, V):
  mesh = plsc.VectorSubcoreMesh(core_axis_name="c", subcore_axis_name="s")
  NC = plsc.get_sparse_core_info().num_cores
  G = B // NW
  EV = E // LANES

  @functools.partial(
      pl.kernel,
      out_type=jax.ShapeDtypeStruct((L, E, B), jnp.float32),
      mesh=mesh,
      compiler_params=pltpu.CompilerParams(use_tc_tiling_on_sc=True,
                                           needs_layout_passes=False),
      scratch_types=[
          pltpu.VMEM((L, G), jnp.int32),
          pltpu.VMEM((NBUF, G), jnp.int32),
          pltpu.VMEM((NBUF, G, 128), jnp.float32),
          pltpu.VMEM((NBUF, E, G), jnp.float32),
          pltpu.VMEM((NBUF, 128), jnp.float32),
          pltpu.SemaphoreType.DMA((NBUF,)),
          pltpu.SemaphoreType.DMA((NBUF,)),
          pltpu.SemaphoreType.DMA((NBUF,)),
      ],
  )
  def k(idx_hbm, tok2_hbm, poss_hbm, out_hbm, idx_v, idx2, rows, chunk,
        posb, sem_p, sem_b, sem_c):
    c = lax.axis_index("c")
    s = lax.axis_index("s")
    wid = s * NC + c
    b0 = wid * G

    pltpu.sync_copy(idx_hbm.at[:, pl.ds(b0, G)], idx_v)

    def step(j, carry):
      @pl.when(j < L)
      def _():
        r = lax.rem(j, NBUF)
        @pl.when(j >= NBUF)
        def _():
          pltpu.make_async_copy(chunk.at[r], out_hbm.at[0, :, pl.ds(0, G)],
                                sem_c.at[r]).wait()
        pltpu.async_copy(poss_hbm.at[j], posb.at[r], sem_p.at[r])
        for k8 in range(G // LANES):
          d = pl.ds(k8 * LANES, LANES)
          idx2[r, d] = lax.shift_right_logical(idx_v[j, d], 1)
        pltpu.async_copy(tok2_hbm.at[idx2.at[r]], rows.at[r], sem_b.at[r])

      jb = j - 2
      @pl.when((jb >= 0) & (jb < L))
      def _():
        r = lax.rem(jb, NBUF)
        pltpu.make_async_copy(poss_hbm.at[0], posb.at[r], sem_p.at[r]).wait()
        pltpu.make_async_copy(tok2_hbm.at[idx2.at[r]], rows.at[r],
                              sem_b.at[r]).wait()
        iot = lax.iota(jnp.int32, LANES)
        zero = jnp.zeros((LANES,), jnp.int32)
        base = [
            (iot + (k8 * LANES)) * 128
            + (idx_v[jb, pl.ds(k8 * LANES, LANES)] & 1) * E
            for k8 in range(G // LANES)
        ]
        bvec = [iot + (k8 * LANES) for k8 in range(G // LANES)]

        @plsc.parallel_loop(0, LANES, 1, unroll=4)
        def _(d):
          rotd = (iot + d) & (LANES - 1)
          for e16 in range(E // LANES):
            ecol = rotd + (e16 * LANES)
            pvec = plsc.load_gather(posb.at[r], [ecol])
            for k8 in range(G // LANES):
              val = plsc.load_gather(rows.at[r], [zero, base[k8] + ecol])
              plsc.store_scatter(chunk.at[r], [ecol, bvec[k8]], val + pvec)

        pltpu.async_copy(chunk.at[r], out_hbm.at[jb, :, pl.ds(b0, G)],
                         sem_c.at[r])

      return carry

    lax.fori_loop(0, L + 2, step, 0)

    for rb in range(NBUF):
      pltpu.make_async_copy(chunk.at[rb], out_hbm.at[0, :, pl.ds(0, G)],
                            sem_c.at[rb]).wait()

  return k


def kernel(in_idx, tok_table, pos_table):
  B, L = in_idx.shape
  V, E = tok_table.shape
  info = plsc.get_sparse_core_info()
  NW = info.num_cores * info.num_subcores
  assert B % (NW * 128) == 0 or B == NW * 128

  idxT = in_idx.T.astype(jnp.int32)
  tok2 = tok_table.reshape(V // 2, 2 * E)
  poss = jnp.pad(pos_table[:L], ((0, 0), (0, 128 - E)))
  out5 = _build(NW, L, B, E, V)(idxT, tok2, poss)
  return out5.transpose(2, 0, 1)
